# Initial kernel scaffold; baseline (speedup 1.0000x reference)
import jax
import jax.numpy as jnp
from jax.experimental import pallas as pl


def kernel(rel_ids, emb_weight):
    raise NotImplementedError("write your pallas kernel here")


# speedup vs baseline: 5.1501x; 5.1501x over previous
import functools

import jax
import jax.numpy as jnp
from jax import lax
from jax.experimental import pallas as pl
from jax.experimental.pallas import tpu as pltpu
from jax.experimental.pallas import tpu_sc as plsc


def _gather_kernel(E, V, D, num_cores, num_subcores):
    NW = num_cores * num_subcores
    b_per_w = E // NW
    CH = 512
    n_chunks = b_per_w // CH
    mesh = plsc.VectorSubcoreMesh(core_axis_name="c", subcore_axis_name="s")

    @functools.partial(
        pl.kernel,
        mesh=mesh,
        compiler_params=pltpu.CompilerParams(use_tc_tiling_on_sc=False),
        out_type=jax.ShapeDtypeStruct((E, D), jnp.float32),
        scratch_types=[
            pltpu.VMEM((CH,), jnp.int32),
            pltpu.VMEM((CH, D), jnp.float32),
            pltpu.SemaphoreType.DMA,
        ],
    )
    def k(idx_hbm, table_hbm, out_hbm, idx_v, rows_v, sem):
        wid = lax.axis_index("s") * num_cores + lax.axis_index("c")
        base = wid * b_per_w

        def body(i, carry):
            start = base + i * CH
            pltpu.sync_copy(idx_hbm.at[pl.ds(start, CH)], idx_v)
            pltpu.async_copy(table_hbm.at[idx_v], rows_v, sem).wait()
            pltpu.sync_copy(rows_v, out_hbm.at[pl.ds(start, CH)])
            return carry

        lax.fori_loop(0, n_chunks, body, 0)

    return k


def kernel(rel_ids, emb_weight):
    E = rel_ids.shape[0]
    V, D = emb_weight.shape
    flat_ids = rel_ids.reshape(-1).astype(jnp.int32)
    info = plsc.get_sparse_core_info()
    k = _gather_kernel(E, V, D, info.num_cores, info.num_subcores)
    return k(flat_ids, emb_weight)


# speedup vs baseline: 5.6180x; 1.0909x over previous
import functools

import jax
import jax.numpy as jnp
from jax import lax
from jax.experimental import pallas as pl
from jax.experimental.pallas import tpu as pltpu
from jax.experimental.pallas import tpu_sc as plsc

_NBUF = 4
_CH = 400


def _gather_kernel(E, V, D, num_cores, num_subcores):
    NW = num_cores * num_subcores
    b_per_w = E // NW
    n_chunks = b_per_w // _CH
    n_outer = n_chunks // _NBUF
    mesh = plsc.VectorSubcoreMesh(core_axis_name="c", subcore_axis_name="s")

    @functools.partial(
        pl.kernel,
        mesh=mesh,
        compiler_params=pltpu.CompilerParams(use_tc_tiling_on_sc=False),
        out_type=jax.ShapeDtypeStruct((E, D), jnp.float32),
        scratch_types=[
            pltpu.VMEM((_NBUF, _CH), jnp.int32),
            pltpu.VMEM((_NBUF, _CH, D), jnp.float32),
        ]
        + [pltpu.SemaphoreType.DMA] * (2 * _NBUF),
    )
    def k(idx_hbm, table_hbm, out_hbm, idx_v, rows_v, *sems):
        gsems = sems[:_NBUF]
        osems = sems[_NBUF:]
        wid = lax.axis_index("s") * num_cores + lax.axis_index("c")
        base = wid * b_per_w

        def gather_start(c, b):
            pltpu.sync_copy(idx_hbm.at[pl.ds(base + c * _CH, _CH)], idx_v.at[b])
            pltpu.make_async_copy(
                table_hbm.at[idx_v.at[b]], rows_v.at[b], gsems[b]
            ).start()

        def gather_wait(c, b):
            pltpu.make_async_copy(
                table_hbm.at[idx_v.at[b]], rows_v.at[b], gsems[b]
            ).wait()

        def write_start(c, b):
            pltpu.make_async_copy(
                rows_v.at[b], out_hbm.at[pl.ds(base + c * _CH, _CH)], osems[b]
            ).start()

        def write_wait(c, b):
            pltpu.make_async_copy(
                rows_v.at[b], out_hbm.at[pl.ds(base + c * _CH, _CH)], osems[b]
            ).wait()

        def step(c, b, start_next, wait_prior):
            bn = (b + 1) % _NBUF
            if start_next:
                if wait_prior:
                    write_wait(c + 1 - _NBUF, bn)
                gather_start(c + 1, bn)
            gather_wait(c, b)
            write_start(c, b)

        gather_start(0, 0)
        for b in range(_NBUF):
            step(b, b, start_next=True, wait_prior=(b + 1 >= _NBUF))

        def body(g, carry):
            c0 = g * _NBUF
            for b in range(_NBUF):
                step(c0 + b, b, start_next=True, wait_prior=True)
            return carry

        lax.fori_loop(1, n_outer - 1, body, 0)

        c0 = (n_outer - 1) * _NBUF
        for b in range(_NBUF):
            step(c0 + b, b, start_next=(b < _NBUF - 1), wait_prior=True)

        for j in range(_NBUF):
            c = n_chunks - _NBUF + j
            write_wait(c, c % _NBUF)

    return k


def kernel(rel_ids, emb_weight):
    E = rel_ids.shape[0]
    V, D = emb_weight.shape
    flat_ids = rel_ids.reshape(-1).astype(jnp.int32)
    info = plsc.get_sparse_core_info()
    k = _gather_kernel(E, V, D, info.num_cores, info.num_subcores)
    return k(flat_ids, emb_weight)
